# baseline (speedup 1.0000x reference)
import functools
import math

import jax
import jax.numpy as jnp
from jax import lax
from jax.experimental import pallas as pl
from jax.experimental.pallas import tpu as pltpu

D = 512
H = 8
DH = 64
FF = 2048
SQRT_D = math.sqrt(512.0)


def _ln(x, g, b):
    m = jnp.mean(x, axis=1, keepdims=True)
    c = x - m
    v = jnp.mean(c * c, axis=1, keepdims=True)
    return c * lax.rsqrt(v + 1e-5) * g + b


def _enc_kernel(tok_ref, emb_ref, pos_ref, wq_ref, wk_ref, wv_ref, wo_ref,
                w1_ref, b1_ref, w2_ref, b2_ref, g1_ref, be1_ref, g2_ref,
                be2_ref, enc_ref, pool_ref):
    T = pos_ref.shape[0]
    tok = tok_ref[0, 0]
    mask_row = (tok != 0).reshape(1, T)
    x = emb_ref[0] * SQRT_D + pos_ref[...]
    h = _ln(x, g1_ref[...], be1_ref[...])
    q = jnp.dot(h, wq_ref[...], preferred_element_type=jnp.float32)
    k = jnp.dot(h, wk_ref[...], preferred_element_type=jnp.float32)
    v = jnp.dot(h, wv_ref[...], preferred_element_type=jnp.float32)
    acc = x
    for hd in range(H):
        s = hd * DH
        qh = q[:, s:s + DH]
        kh = k[:, s:s + DH]
        vh = v[:, s:s + DH]
        sc = lax.dot_general(qh, kh, (((1,), (1,)), ((), ())),
                             preferred_element_type=jnp.float32) * 0.125
        sc = jnp.where(mask_row, sc, -1e9)
        mx = jnp.max(sc, axis=1, keepdims=True)
        ex = jnp.exp(sc - mx)
        p = ex / jnp.sum(ex, axis=1, keepdims=True)
        oh = jnp.dot(p, vh, preferred_element_type=jnp.float32)
        acc = acc + jnp.dot(oh, wo_ref[s:s + DH, :],
                            preferred_element_type=jnp.float32)
    h2 = _ln(acc, g2_ref[...], be2_ref[...])
    f = jnp.maximum(
        jnp.dot(h2, w1_ref[...], preferred_element_type=jnp.float32)
        + b1_ref[...], 0.0)
    out = acc + jnp.dot(f, w2_ref[...],
                        preferred_element_type=jnp.float32) + b2_ref[...]
    enc_m = out * jnp.transpose(mask_row.astype(jnp.float32))
    enc_ref[0] = enc_m
    ln_len = jnp.maximum(jnp.sum(mask_row.astype(jnp.float32)), 1.0)
    pool_ref[0] = jnp.sum(enc_m, axis=0, keepdims=True) / jnp.sqrt(
        jnp.float32(D) * ln_len)


def _encode(tokens3, emb_rows, pos_t, p, B, T):
    const = lambda i: (0, 0)
    enc, pool = pl.pallas_call(
        _enc_kernel,
        grid=(B,),
        in_specs=[
            pl.BlockSpec((1, 1, T), lambda i: (i, 0, 0)),
            pl.BlockSpec((1, T, D), lambda i: (i, 0, 0)),
            pl.BlockSpec((T, D), const),
            pl.BlockSpec((D, D), const),
            pl.BlockSpec((D, D), const),
            pl.BlockSpec((D, D), const),
            pl.BlockSpec((D, D), const),
            pl.BlockSpec((D, FF), const),
            pl.BlockSpec((1, FF), const),
            pl.BlockSpec((FF, D), const),
            pl.BlockSpec((1, D), const),
            pl.BlockSpec((1, D), const),
            pl.BlockSpec((1, D), const),
            pl.BlockSpec((1, D), const),
            pl.BlockSpec((1, D), const),
        ],
        out_specs=[
            pl.BlockSpec((1, T, D), lambda i: (i, 0, 0)),
            pl.BlockSpec((1, 1, D), lambda i: (i, 0, 0)),
        ],
        out_shape=[
            jax.ShapeDtypeStruct((B, T, D), jnp.float32),
            jax.ShapeDtypeStruct((B, 1, D), jnp.float32),
        ],
    )(tokens3, emb_rows, pos_t,
      p['Wq'], p['Wk'], p['Wv'], p['Wo'],
      p['W1'], p['b1'].reshape(1, FF), p['W2'], p['b2'].reshape(1, D),
      p['g1'].reshape(1, D), p['be1'].reshape(1, D),
      p['g2'].reshape(1, D), p['be2'].reshape(1, D))
    return enc, pool


def _select_kernel(ku_ref, cu_ref, ckm_ref, cs_ref, use_ref, sel_ref,
                   cka_ref):
    N, K, _ = ku_ref.shape
    ku = ku_ref[...]
    cu = cu_ref[...]
    ck = jnp.sum(ku * cu[:, None, :], axis=2)
    valid = ckm_ref[...] != 0
    ckm = jnp.where(valid, ck, -1e9)
    mx = jnp.max(ckm, axis=1, keepdims=True)
    iota = lax.broadcasted_iota(jnp.int32, (N, K), 1)
    am = jnp.min(jnp.where(ckm == mx, iota, K), axis=1, keepdims=True)
    ids = jnp.where(use_ref[0, 0] != 0, cs_ref[...], am)
    iota_n = lax.broadcasted_iota(jnp.int32, (N, N), 1)
    oh = (ids == iota_n).astype(jnp.float32)
    sel_f = lax.dot_general(oh, ids.astype(jnp.float32),
                            (((1,), (0,)), ((), ())),
                            preferred_element_type=jnp.float32)
    sel_ref[...] = sel_f.astype(jnp.int32)
    cka_ref[...] = jnp.where(valid, ck, 0.0)


def _select(know_use, ctx_use, ck_mask_i, cs_ids2, use_flag):
    N, K, _ = know_use.shape
    sel, cka = pl.pallas_call(
        _select_kernel,
        grid=(1,),
        in_specs=[
            pl.BlockSpec((N, K, D), lambda i: (0, 0, 0)),
            pl.BlockSpec((N, D), lambda i: (0, 0)),
            pl.BlockSpec((N, K), lambda i: (0, 0)),
            pl.BlockSpec((N, 1), lambda i: (0, 0)),
            pl.BlockSpec((1, 1), lambda i: (0, 0)),
        ],
        out_specs=[
            pl.BlockSpec((N, 1), lambda i: (0, 0)),
            pl.BlockSpec((N, K), lambda i: (0, 0)),
        ],
        out_shape=[
            jax.ShapeDtypeStruct((N, 1), jnp.int32),
            jax.ShapeDtypeStruct((N, K), jnp.float32),
        ],
    )(know_use, ctx_use, ck_mask_i, cs_ids2, use_flag)
    return sel, cka


def _gc_kernel(sel_ref, kenc_ref, ktok_ref, cenc_ref, stok_ref, enc_ref,
               mask_ref):
    Tk = kenc_ref.shape[2]
    enc_ref[0, :Tk] = kenc_ref[0, 0]
    enc_ref[0, Tk:] = cenc_ref[0]
    mask_ref[0, 0, :Tk] = (ktok_ref[0, 0, 0] != 0).astype(jnp.int32)
    mask_ref[0, 0, Tk:] = (stok_ref[0, 0] != 0).astype(jnp.int32)


def _gather_concat(sel, know_enc4, know_tok4, ctx_enc, src_tok3):
    N, K, Tk, _ = know_enc4.shape
    Ts = ctx_enc.shape[1]
    Tf = Tk + Ts
    grid_spec = pltpu.PrefetchScalarGridSpec(
        num_scalar_prefetch=1,
        grid=(N,),
        in_specs=[
            pl.BlockSpec((1, 1, Tk, D), lambda n, s: (n, s[n], 0, 0)),
            pl.BlockSpec((1, 1, 1, Tk), lambda n, s: (n, s[n], 0, 0)),
            pl.BlockSpec((1, Ts, D), lambda n, s: (n, 0, 0)),
            pl.BlockSpec((1, 1, Ts), lambda n, s: (n, 0, 0)),
        ],
        out_specs=[
            pl.BlockSpec((1, Tf, D), lambda n, s: (n, 0, 0)),
            pl.BlockSpec((1, 1, Tf), lambda n, s: (n, 0, 0)),
        ],
    )
    enc, mask_i = pl.pallas_call(
        _gc_kernel,
        grid_spec=grid_spec,
        out_shape=[
            jax.ShapeDtypeStruct((N, Tf, D), jnp.float32),
            jax.ShapeDtypeStruct((N, 1, Tf), jnp.int32),
        ],
    )(sel, know_enc4, know_tok4, ctx_enc, src_tok3)
    return enc, mask_i


def _embed_rows(table, idx):
    return jnp.take(table, idx, axis=0)


def kernel(src_tokens, know_tokens, ck_mask, cs_ids, use_cs_ids, params):
    p = params
    src = src_tokens.astype(jnp.int32)
    know = know_tokens.astype(jnp.int32)
    N, Ts = src.shape
    _, K, Tk = know.shape
    idx = jnp.concatenate([src.reshape(-1), know.reshape(-1)])
    rows = _embed_rows(p['emb'], idx)
    ctx_rows = rows[:N * Ts].reshape(N, Ts, D)
    know_rows = rows[N * Ts:].reshape(N * K, Tk, D)
    ctx_enc, ctx_pool = _encode(src.reshape(N, 1, Ts), ctx_rows,
                                p['pos'][:Ts], p, N, Ts)
    know_enc, know_pool = _encode(know.reshape(N * K, 1, Tk), know_rows,
                                  p['pos'][:Tk], p, N * K, Tk)
    use_flag = jnp.asarray(use_cs_ids, jnp.int32).reshape(1, 1)
    sel2, cka = _select(know_pool.reshape(N, K, D), ctx_pool.reshape(N, D),
                        ck_mask.astype(jnp.int32), cs_ids.astype(
                            jnp.int32).reshape(N, 1), use_flag)
    full_enc, full_mask_i = _gather_concat(
        sel2.reshape(N), know_enc.reshape(N, K, Tk, D),
        know.reshape(N, K, 1, Tk), ctx_enc, src.reshape(N, 1, Ts))
    return full_enc, full_mask_i.reshape(N, Tk + Ts) != 0, cka


# speedup vs baseline: 1.7693x; 1.7693x over previous
import functools
import math

import jax
import jax.numpy as jnp
from jax import lax
from jax.experimental import pallas as pl
from jax.experimental.pallas import tpu as pltpu

D = 512
H = 8
DH = 64
FF = 2048
SQRT_D = math.sqrt(512.0)


def _ln(x, g, b):
    m = jnp.mean(x, axis=1, keepdims=True)
    c = x - m
    v = jnp.mean(c * c, axis=1, keepdims=True)
    return c * lax.rsqrt(v + 1e-5) * g + b


def _enc_kernel(S, T, tok_ref, emb_ref, pos_ref, wq_ref, wk_ref, wv_ref,
                wo_ref, w1_ref, b1_ref, w2_ref, b2_ref, g1_ref, be1_ref,
                g2_ref, be2_ref, enc_ref, pool_ref):
    R = S * T
    tok = tok_ref[0, 0]
    mask_row = (tok != 0).reshape(1, R)
    if S == 1:
        x = emb_ref[0] * SQRT_D + pos_ref[...]
        bias = jnp.where(mask_row, 0.0, -1e9)
    else:
        x = (emb_ref[0].reshape(S, T, D) * SQRT_D
             + pos_ref[...][None]).reshape(R, D)
        ri = lax.broadcasted_iota(jnp.int32, (R, R), 0) // T
        ci = lax.broadcasted_iota(jnp.int32, (R, R), 1) // T
        seg = (ri == ci) & mask_row
        bias = jnp.where(seg, 0.0, -1e9)
    h = _ln(x, g1_ref[...], be1_ref[...])
    q = jnp.dot(h, wq_ref[...], preferred_element_type=jnp.float32) * 0.125
    k = jnp.dot(h, wk_ref[...], preferred_element_type=jnp.float32)
    v = jnp.dot(h, wv_ref[...], preferred_element_type=jnp.float32)
    acc = x
    for hd in range(H):
        s = hd * DH
        sc = lax.dot_general(q[:, s:s + DH], k[:, s:s + DH],
                             (((1,), (1,)), ((), ())),
                             preferred_element_type=jnp.float32) + bias
        mx = jnp.max(sc, axis=1, keepdims=True)
        ex = jnp.exp(sc - mx)
        p = ex / jnp.sum(ex, axis=1, keepdims=True)
        oh = jnp.dot(p, v[:, s:s + DH], preferred_element_type=jnp.float32)
        acc = acc + jnp.dot(oh, wo_ref[s:s + DH, :],
                            preferred_element_type=jnp.float32)
    h2 = _ln(acc, g2_ref[...], be2_ref[...])
    f = jnp.maximum(
        jnp.dot(h2, w1_ref[...], preferred_element_type=jnp.float32)
        + b1_ref[...], 0.0)
    out = acc + jnp.dot(f, w2_ref[...],
                        preferred_element_type=jnp.float32) + b2_ref[...]
    enc_m = out * jnp.transpose(mask_row.astype(jnp.float32))
    enc_ref[0] = enc_m
    lens = jnp.sum(mask_row.astype(jnp.float32).reshape(S, T), axis=1,
                   keepdims=True)
    pool = jnp.sum(enc_m.reshape(S, T, D), axis=1)
    pool_ref[0] = pool * lax.rsqrt(
        jnp.float32(D) * jnp.maximum(lens, 1.0))


def _encode(tokens3, emb_rows, pos_t, p, B, T, S):
    G = B // S
    R = S * T
    const = lambda i: (0, 0)
    body = functools.partial(_enc_kernel, S, T)
    enc, pool = pl.pallas_call(
        body,
        grid=(G,),
        in_specs=[
            pl.BlockSpec((1, 1, R), lambda i: (i, 0, 0)),
            pl.BlockSpec((1, R, D), lambda i: (i, 0, 0)),
            pl.BlockSpec((T, D), const),
            pl.BlockSpec((D, D), const),
            pl.BlockSpec((D, D), const),
            pl.BlockSpec((D, D), const),
            pl.BlockSpec((D, D), const),
            pl.BlockSpec((D, FF), const),
            pl.BlockSpec((1, FF), const),
            pl.BlockSpec((FF, D), const),
            pl.BlockSpec((1, D), const),
            pl.BlockSpec((1, D), const),
            pl.BlockSpec((1, D), const),
            pl.BlockSpec((1, D), const),
            pl.BlockSpec((1, D), const),
        ],
        out_specs=[
            pl.BlockSpec((1, R, D), lambda i: (i, 0, 0)),
            pl.BlockSpec((1, S, D), lambda i: (i, 0, 0)),
        ],
        out_shape=[
            jax.ShapeDtypeStruct((G, R, D), jnp.float32),
            jax.ShapeDtypeStruct((G, S, D), jnp.float32),
        ],
    )(tokens3, emb_rows, pos_t,
      p['Wq'], p['Wk'], p['Wv'], p['Wo'],
      p['W1'], p['b1'].reshape(1, FF), p['W2'], p['b2'].reshape(1, D),
      p['g1'].reshape(1, D), p['be1'].reshape(1, D),
      p['g2'].reshape(1, D), p['be2'].reshape(1, D))
    return enc, pool


def _select_kernel(ku_ref, cu_ref, ckm_ref, cs_ref, use_ref, sel_ref,
                   cka_ref):
    N, K, _ = ku_ref.shape
    ku = ku_ref[...]
    cu = cu_ref[...]
    ck = jnp.sum(ku * cu[:, None, :], axis=2)
    valid = ckm_ref[...] != 0
    ckm = jnp.where(valid, ck, -1e9)
    mx = jnp.max(ckm, axis=1, keepdims=True)
    iota = lax.broadcasted_iota(jnp.int32, (N, K), 1)
    am = jnp.min(jnp.where(ckm == mx, iota, K), axis=1, keepdims=True)
    ids = jnp.where(use_ref[0, 0] != 0, cs_ref[...], am)
    iota_n = lax.broadcasted_iota(jnp.int32, (N, N), 1)
    oh = (ids == iota_n).astype(jnp.float32)
    sel_f = lax.dot_general(oh, ids.astype(jnp.float32),
                            (((1,), (0,)), ((), ())),
                            preferred_element_type=jnp.float32)
    sel_ref[...] = sel_f.astype(jnp.int32)
    cka_ref[...] = jnp.where(valid, ck, 0.0)


def _select(know_use, ctx_use, ck_mask_i, cs_ids2, use_flag):
    N, K, _ = know_use.shape
    sel, cka = pl.pallas_call(
        _select_kernel,
        grid=(1,),
        in_specs=[
            pl.BlockSpec((N, K, D), lambda i: (0, 0, 0)),
            pl.BlockSpec((N, D), lambda i: (0, 0)),
            pl.BlockSpec((N, K), lambda i: (0, 0)),
            pl.BlockSpec((N, 1), lambda i: (0, 0)),
            pl.BlockSpec((1, 1), lambda i: (0, 0)),
        ],
        out_specs=[
            pl.BlockSpec((N, 1), lambda i: (0, 0)),
            pl.BlockSpec((N, K), lambda i: (0, 0)),
        ],
        out_shape=[
            jax.ShapeDtypeStruct((N, 1), jnp.int32),
            jax.ShapeDtypeStruct((N, K), jnp.float32),
        ],
    )(know_use, ctx_use, ck_mask_i, cs_ids2, use_flag)
    return sel, cka


def _gc_kernel(sel_ref, kenc_ref, ktok_ref, cenc_ref, stok_ref, enc_ref,
               mask_ref):
    Tk = kenc_ref.shape[2]
    enc_ref[0, :Tk] = kenc_ref[0, 0]
    enc_ref[0, Tk:] = cenc_ref[0]
    mask_ref[0, 0, :Tk] = (ktok_ref[0, 0, 0] != 0).astype(jnp.int32)
    mask_ref[0, 0, Tk:] = (stok_ref[0, 0] != 0).astype(jnp.int32)


def _gather_concat(sel, know_enc4, know_tok4, ctx_enc, src_tok3):
    N, K, Tk, _ = know_enc4.shape
    Ts = ctx_enc.shape[1]
    Tf = Tk + Ts
    grid_spec = pltpu.PrefetchScalarGridSpec(
        num_scalar_prefetch=1,
        grid=(N,),
        in_specs=[
            pl.BlockSpec((1, 1, Tk, D), lambda n, s: (n, s[n], 0, 0)),
            pl.BlockSpec((1, 1, 1, Tk), lambda n, s: (n, s[n], 0, 0)),
            pl.BlockSpec((1, Ts, D), lambda n, s: (n, 0, 0)),
            pl.BlockSpec((1, 1, Ts), lambda n, s: (n, 0, 0)),
        ],
        out_specs=[
            pl.BlockSpec((1, Tf, D), lambda n, s: (n, 0, 0)),
            pl.BlockSpec((1, 1, Tf), lambda n, s: (n, 0, 0)),
        ],
    )
    enc, mask_i = pl.pallas_call(
        _gc_kernel,
        grid_spec=grid_spec,
        out_shape=[
            jax.ShapeDtypeStruct((N, Tf, D), jnp.float32),
            jax.ShapeDtypeStruct((N, 1, Tf), jnp.int32),
        ],
    )(sel, know_enc4, know_tok4, ctx_enc, src_tok3)
    return enc, mask_i


def _embed_rows(table, idx):
    return jnp.take(table, idx, axis=0)


def kernel(src_tokens, know_tokens, ck_mask, cs_ids, use_cs_ids, params):
    p = params
    src = src_tokens.astype(jnp.int32)
    know = know_tokens.astype(jnp.int32)
    N, Ts = src.shape
    _, K, Tk = know.shape
    idx = jnp.concatenate([src.reshape(-1), know.reshape(-1)])
    rows = _embed_rows(p['emb'], idx)
    S_K = 4
    ctx_rows = rows[:N * Ts].reshape(N, Ts, D)
    know_rows = rows[N * Ts:].reshape(N * K // S_K, S_K * Tk, D)
    ctx_enc, ctx_pool = _encode(src.reshape(N, 1, Ts), ctx_rows,
                                p['pos'][:Ts], p, N, Ts, 1)
    know_enc, know_pool = _encode(
        know.reshape(N * K // S_K, 1, S_K * Tk), know_rows,
        p['pos'][:Tk], p, N * K, Tk, S_K)
    use_flag = jnp.asarray(use_cs_ids, jnp.int32).reshape(1, 1)
    sel2, cka = _select(know_pool.reshape(N, K, D), ctx_pool.reshape(N, D),
                        ck_mask.astype(jnp.int32), cs_ids.astype(
                            jnp.int32).reshape(N, 1), use_flag)
    full_enc, full_mask_i = _gather_concat(
        sel2.reshape(N), know_enc.reshape(N, K, Tk, D),
        know.reshape(N, K, 1, Tk), ctx_enc, src.reshape(N, 1, Ts))
    return full_enc, full_mask_i.reshape(N, Tk + Ts) != 0, cka


# speedup vs baseline: 1.8509x; 1.0462x over previous
import functools
import math

import jax
import jax.numpy as jnp
from jax import lax
from jax.experimental import pallas as pl
from jax.experimental.pallas import tpu as pltpu
from jax.experimental.pallas import tpu_sc as plsc

D = 512
H = 8
DH = 64
FF = 2048
SQRT_D = math.sqrt(512.0)


def _ln(x, g, b):
    m = jnp.mean(x, axis=1, keepdims=True)
    c = x - m
    v = jnp.mean(c * c, axis=1, keepdims=True)
    return c * lax.rsqrt(v + 1e-5) * g + b


def _enc_kernel(S, T, tok_ref, emb_ref, pos_ref, wq_ref, wk_ref, wv_ref,
                wo_ref, w1_ref, b1_ref, w2_ref, b2_ref, g1_ref, be1_ref,
                g2_ref, be2_ref, enc_ref, pool_ref):
    R = S * T
    tok = tok_ref[0, 0]
    mask_row = (tok != 0).reshape(1, R)
    if S == 1:
        x = emb_ref[0] * SQRT_D + pos_ref[...]
        bias = jnp.where(mask_row, 0.0, -1e9)
    else:
        x = (emb_ref[0].reshape(S, T, D) * SQRT_D
             + pos_ref[...][None]).reshape(R, D)
        ri = lax.broadcasted_iota(jnp.int32, (R, R), 0) // T
        ci = lax.broadcasted_iota(jnp.int32, (R, R), 1) // T
        seg = (ri == ci) & mask_row
        bias = jnp.where(seg, 0.0, -1e9)
    h = _ln(x, g1_ref[...], be1_ref[...])
    q = jnp.dot(h, wq_ref[...], preferred_element_type=jnp.float32) * 0.125
    k = jnp.dot(h, wk_ref[...], preferred_element_type=jnp.float32)
    v = jnp.dot(h, wv_ref[...], preferred_element_type=jnp.float32)
    acc = x
    for hd in range(H):
        s = hd * DH
        sc = lax.dot_general(q[:, s:s + DH], k[:, s:s + DH],
                             (((1,), (1,)), ((), ())),
                             preferred_element_type=jnp.float32) + bias
        mx = jnp.max(sc, axis=1, keepdims=True)
        ex = jnp.exp(sc - mx)
        p = ex / jnp.sum(ex, axis=1, keepdims=True)
        oh = jnp.dot(p, v[:, s:s + DH], preferred_element_type=jnp.float32)
        acc = acc + jnp.dot(oh, wo_ref[s:s + DH, :],
                            preferred_element_type=jnp.float32)
    h2 = _ln(acc, g2_ref[...], be2_ref[...])
    f = jnp.maximum(
        jnp.dot(h2, w1_ref[...], preferred_element_type=jnp.float32)
        + b1_ref[...], 0.0)
    out = acc + jnp.dot(f, w2_ref[...],
                        preferred_element_type=jnp.float32) + b2_ref[...]
    enc_m = out * jnp.transpose(mask_row.astype(jnp.float32))
    enc_ref[0] = enc_m
    lens = jnp.sum(mask_row.astype(jnp.float32).reshape(S, T), axis=1,
                   keepdims=True)
    pool = jnp.sum(enc_m.reshape(S, T, D), axis=1)
    pool_ref[0] = pool * lax.rsqrt(
        jnp.float32(D) * jnp.maximum(lens, 1.0))


def _encode(tokens3, emb_rows, pos_t, p, B, T, S):
    G = B // S
    R = S * T
    const = lambda i: (0, 0)
    body = functools.partial(_enc_kernel, S, T)
    enc, pool = pl.pallas_call(
        body,
        grid=(G,),
        in_specs=[
            pl.BlockSpec((1, 1, R), lambda i: (i, 0, 0)),
            pl.BlockSpec((1, R, D), lambda i: (i, 0, 0)),
            pl.BlockSpec((T, D), const),
            pl.BlockSpec((D, D), const),
            pl.BlockSpec((D, D), const),
            pl.BlockSpec((D, D), const),
            pl.BlockSpec((D, D), const),
            pl.BlockSpec((D, FF), const),
            pl.BlockSpec((1, FF), const),
            pl.BlockSpec((FF, D), const),
            pl.BlockSpec((1, D), const),
            pl.BlockSpec((1, D), const),
            pl.BlockSpec((1, D), const),
            pl.BlockSpec((1, D), const),
            pl.BlockSpec((1, D), const),
        ],
        out_specs=[
            pl.BlockSpec((1, R, D), lambda i: (i, 0, 0)),
            pl.BlockSpec((1, S, D), lambda i: (i, 0, 0)),
        ],
        out_shape=[
            jax.ShapeDtypeStruct((G, R, D), jnp.float32),
            jax.ShapeDtypeStruct((G, S, D), jnp.float32),
        ],
    )(tokens3, emb_rows, pos_t,
      p['Wq'], p['Wk'], p['Wv'], p['Wo'],
      p['W1'], p['b1'].reshape(1, FF), p['W2'], p['b2'].reshape(1, D),
      p['g1'].reshape(1, D), p['be1'].reshape(1, D),
      p['g2'].reshape(1, D), p['be2'].reshape(1, D))
    return enc, pool


def _select_kernel(ku_ref, cu_ref, ckm_ref, cs_ref, use_ref, sel_ref,
                   cka_ref):
    N, K, _ = ku_ref.shape
    ku = ku_ref[...]
    cu = cu_ref[...]
    ck = jnp.sum(ku * cu[:, None, :], axis=2)
    valid = ckm_ref[...] != 0
    ckm = jnp.where(valid, ck, -1e9)
    mx = jnp.max(ckm, axis=1, keepdims=True)
    iota = lax.broadcasted_iota(jnp.int32, (N, K), 1)
    am = jnp.min(jnp.where(ckm == mx, iota, K), axis=1, keepdims=True)
    ids = jnp.where(use_ref[0, 0] != 0, cs_ref[...], am)
    iota_n = lax.broadcasted_iota(jnp.int32, (N, N), 1)
    oh = (ids == iota_n).astype(jnp.float32)
    sel_f = lax.dot_general(oh, ids.astype(jnp.float32),
                            (((1,), (0,)), ((), ())),
                            preferred_element_type=jnp.float32)
    sel_ref[...] = sel_f.astype(jnp.int32)
    cka_ref[...] = jnp.where(valid, ck, 0.0)


def _select(know_use, ctx_use, ck_mask_i, cs_ids2, use_flag):
    N, K, _ = know_use.shape
    sel, cka = pl.pallas_call(
        _select_kernel,
        grid=(1,),
        in_specs=[
            pl.BlockSpec((N, K, D), lambda i: (0, 0, 0)),
            pl.BlockSpec((N, D), lambda i: (0, 0)),
            pl.BlockSpec((N, K), lambda i: (0, 0)),
            pl.BlockSpec((N, 1), lambda i: (0, 0)),
            pl.BlockSpec((1, 1), lambda i: (0, 0)),
        ],
        out_specs=[
            pl.BlockSpec((N, 1), lambda i: (0, 0)),
            pl.BlockSpec((N, K), lambda i: (0, 0)),
        ],
        out_shape=[
            jax.ShapeDtypeStruct((N, 1), jnp.int32),
            jax.ShapeDtypeStruct((N, K), jnp.float32),
        ],
    )(know_use, ctx_use, ck_mask_i, cs_ids2, use_flag)
    return sel, cka


def _gc_kernel(sel_ref, kenc_ref, ktok_ref, cenc_ref, stok_ref, enc_ref,
               mask_ref):
    Tk = kenc_ref.shape[2]
    enc_ref[0, :Tk] = kenc_ref[0, 0]
    enc_ref[0, Tk:] = cenc_ref[0]
    mask_ref[0, 0, :Tk] = (ktok_ref[0, 0, 0] != 0).astype(jnp.int32)
    mask_ref[0, 0, Tk:] = (stok_ref[0, 0] != 0).astype(jnp.int32)


def _gather_concat(sel, know_enc4, know_tok4, ctx_enc, src_tok3):
    N, K, Tk, _ = know_enc4.shape
    Ts = ctx_enc.shape[1]
    Tf = Tk + Ts
    grid_spec = pltpu.PrefetchScalarGridSpec(
        num_scalar_prefetch=1,
        grid=(N,),
        in_specs=[
            pl.BlockSpec((1, 1, Tk, D), lambda n, s: (n, s[n], 0, 0)),
            pl.BlockSpec((1, 1, 1, Tk), lambda n, s: (n, s[n], 0, 0)),
            pl.BlockSpec((1, Ts, D), lambda n, s: (n, 0, 0)),
            pl.BlockSpec((1, 1, Ts), lambda n, s: (n, 0, 0)),
        ],
        out_specs=[
            pl.BlockSpec((1, Tf, D), lambda n, s: (n, 0, 0)),
            pl.BlockSpec((1, 1, Tf), lambda n, s: (n, 0, 0)),
        ],
    )
    enc, mask_i = pl.pallas_call(
        _gc_kernel,
        grid_spec=grid_spec,
        out_shape=[
            jax.ShapeDtypeStruct((N, Tf, D), jnp.float32),
            jax.ShapeDtypeStruct((N, 1, Tf), jnp.int32),
        ],
    )(sel, know_enc4, know_tok4, ctx_enc, src_tok3)
    return enc, mask_i


def _embed_rows(table, idx):
    B = idx.shape[0]
    info = plsc.get_sparse_core_info()
    NC, NS = info.num_cores, info.num_subcores
    NW = NC * NS
    b_per_w = B // NW
    CH = 64
    nch = b_per_w // CH

    mesh = plsc.VectorSubcoreMesh(core_axis_name="c", subcore_axis_name="s")

    @functools.partial(
        pl.kernel, mesh=mesh,
        out_type=jax.ShapeDtypeStruct((B, D), jnp.float32),
        scratch_types=[
            pltpu.VMEM((b_per_w,), jnp.int32),
            pltpu.VMEM((CH, D), jnp.float32),
            pltpu.VMEM((CH, D), jnp.float32),
            pltpu.SemaphoreType.DMA,
            pltpu.SemaphoreType.DMA,
            pltpu.SemaphoreType.DMA,
        ],
    )
    def gk(table_hbm, idx_hbm, out_hbm, idx_v, rows_a, rows_b, sem_a, sem_b,
           sem_out):
        wid = lax.axis_index("s") * NC + lax.axis_index("c")
        base = wid * b_per_w
        pltpu.sync_copy(idx_hbm.at[pl.ds(base, b_per_w)], idx_v)
        bufs = (rows_a, rows_b)
        sems = (sem_a, sem_b)
        copies = []
        for c in range(nch):
            copies.append(pltpu.async_copy(
                table_hbm.at[idx_v.at[pl.ds(c * CH, CH)]],
                bufs[c % 2], sems[c % 2]))
            if c > 0:
                copies[c - 1].wait()
                pltpu.async_copy(bufs[(c - 1) % 2],
                                 out_hbm.at[pl.ds(base + (c - 1) * CH, CH)],
                                 sem_out).wait()
        copies[nch - 1].wait()
        pltpu.async_copy(bufs[(nch - 1) % 2],
                         out_hbm.at[pl.ds(base + (nch - 1) * CH, CH)],
                         sem_out).wait()

    return gk(table, idx)


def kernel(src_tokens, know_tokens, ck_mask, cs_ids, use_cs_ids, params):
    p = params
    src = src_tokens.astype(jnp.int32)
    know = know_tokens.astype(jnp.int32)
    N, Ts = src.shape
    _, K, Tk = know.shape
    idx = jnp.concatenate([src.reshape(-1), know.reshape(-1)])
    rows = _embed_rows(p['emb'], idx)
    S_K = 4
    ctx_rows = rows[:N * Ts].reshape(N, Ts, D)
    know_rows = rows[N * Ts:].reshape(N * K // S_K, S_K * Tk, D)
    ctx_enc, ctx_pool = _encode(src.reshape(N, 1, Ts), ctx_rows,
                                p['pos'][:Ts], p, N, Ts, 1)
    know_enc, know_pool = _encode(
        know.reshape(N * K // S_K, 1, S_K * Tk), know_rows,
        p['pos'][:Tk], p, N * K, Tk, S_K)
    use_flag = jnp.asarray(use_cs_ids, jnp.int32).reshape(1, 1)
    sel2, cka = _select(know_pool.reshape(N, K, D), ctx_pool.reshape(N, D),
                        ck_mask.astype(jnp.int32), cs_ids.astype(
                            jnp.int32).reshape(N, 1), use_flag)
    full_enc, full_mask_i = _gather_concat(
        sel2.reshape(N), know_enc.reshape(N, K, Tk, D),
        know.reshape(N, K, 1, Tk), ctx_enc, src.reshape(N, 1, Ts))
    return full_enc, full_mask_i.reshape(N, Tk + Ts) != 0, cka


# speedup vs baseline: 2.3026x; 1.2440x over previous
import functools
import math

import jax
import jax.numpy as jnp
from jax import lax
from jax.experimental import pallas as pl
from jax.experimental.pallas import tpu as pltpu
from jax.experimental.pallas import tpu_sc as plsc

D = 512
H = 8
DH = 64
FF = 2048
SQRT_D = math.sqrt(512.0)


def _fdot(a, b):
    return jnp.dot(a, b, preferred_element_type=jnp.float32)


def _fdot_t(a, b):
    return lax.dot_general(a, b, (((1,), (1,)), ((), ())),
                           preferred_element_type=jnp.float32)


def _ln(x, g, b):
    m = jnp.mean(x, axis=1, keepdims=True)
    c = x - m
    v = jnp.mean(c * c, axis=1, keepdims=True)
    return c * lax.rsqrt(v + 1e-5) * g + b


def _enc_kernel(S, T, *refs):
    if S == 1:
        (tok_ref, emb_ref, pos_ref, wq_ref, wk_ref, wv_ref,
         wo_ref, w1_ref, b1_ref, w2_ref, b2_ref, g1_ref, be1_ref,
         g2_ref, be2_ref, enc_ref, pool_ref) = refs
        segb_ref = None
    else:
        (tok_ref, emb_ref, pos_ref, segb_ref, wq_ref, wk_ref, wv_ref,
         wo_ref, w1_ref, b1_ref, w2_ref, b2_ref, g1_ref, be1_ref,
         g2_ref, be2_ref, enc_ref, pool_ref) = refs
    R = S * T
    tok = tok_ref[0, 0]
    mask_row = (tok != 0).reshape(1, R)
    kb = jnp.where(mask_row, 0.0, -1e9)
    if S == 1:
        x = emb_ref[0] * SQRT_D + pos_ref[...]
        bias = kb
    else:
        x = (emb_ref[0].reshape(S, T, D) * SQRT_D
             + pos_ref[...][None]).reshape(R, D)
        bias = segb_ref[...] + kb
    h = _ln(x, g1_ref[...], be1_ref[...])
    q = _fdot(h, wq_ref[...]) * 0.125
    k = _fdot(h, wk_ref[...])
    v = _fdot(h, wv_ref[...])
    acc = x
    for hd in range(H):
        s = hd * DH
        sc = _fdot_t(q[:, s:s + DH], k[:, s:s + DH]) + bias
        ex = jnp.exp(sc)
        rs = 1.0 / jnp.sum(ex, axis=1, keepdims=True)
        oh = _fdot(ex, v[:, s:s + DH]) * rs
        acc = acc + _fdot(oh, wo_ref[s:s + DH, :])
    h2 = _ln(acc, g2_ref[...], be2_ref[...])
    f = jnp.maximum(_fdot(h2, w1_ref[...]) + b1_ref[...], 0.0)
    out = acc + _fdot(f, w2_ref[...]) + b2_ref[...]
    enc_m = out * jnp.transpose(mask_row.astype(jnp.float32))
    enc_ref[0] = enc_m
    lens = jnp.sum(mask_row.astype(jnp.float32).reshape(S, T), axis=1,
                   keepdims=True)
    pool = jnp.sum(enc_m.reshape(S, T, D), axis=1)
    pool_ref[0] = pool * lax.rsqrt(
        jnp.float32(D) * jnp.maximum(lens, 1.0))


def _encode(tokens3, emb_rows, pos_t, p, B, T, S):
    G = B // S
    R = S * T
    const = lambda i: (0, 0)
    body = functools.partial(_enc_kernel, S, T)
    seg_specs = []
    seg_ins = []
    if S > 1:
        seg_specs = [pl.BlockSpec((R, R), const)]
        ri = lax.broadcasted_iota(jnp.int32, (R, R), 0) // T
        ci = lax.broadcasted_iota(jnp.int32, (R, R), 1) // T
        seg_ins = [jnp.where(ri == ci, 0.0, -1e9).astype(jnp.float32)]
    enc, pool = pl.pallas_call(
        body,
        grid=(G,),
        in_specs=[
            pl.BlockSpec((1, 1, R), lambda i: (i, 0, 0)),
            pl.BlockSpec((1, R, D), lambda i: (i, 0, 0)),
            pl.BlockSpec((T, D), const),
        ] + seg_specs + [
            pl.BlockSpec((D, D), const),
            pl.BlockSpec((D, D), const),
            pl.BlockSpec((D, D), const),
            pl.BlockSpec((D, D), const),
            pl.BlockSpec((D, FF), const),
            pl.BlockSpec((1, FF), const),
            pl.BlockSpec((FF, D), const),
            pl.BlockSpec((1, D), const),
            pl.BlockSpec((1, D), const),
            pl.BlockSpec((1, D), const),
            pl.BlockSpec((1, D), const),
            pl.BlockSpec((1, D), const),
        ],
        out_specs=[
            pl.BlockSpec((1, R, D), lambda i: (i, 0, 0)),
            pl.BlockSpec((1, S, D), lambda i: (i, 0, 0)),
        ],
        out_shape=[
            jax.ShapeDtypeStruct((G, R, D), jnp.float32),
            jax.ShapeDtypeStruct((G, S, D), jnp.float32),
        ],
    )(tokens3, emb_rows, pos_t, *seg_ins,
      p['Wq'], p['Wk'], p['Wv'], p['Wo'],
      p['W1'], p['b1'].reshape(1, FF), p['W2'], p['b2'].reshape(1, D),
      p['g1'].reshape(1, D), p['be1'].reshape(1, D),
      p['g2'].reshape(1, D), p['be2'].reshape(1, D))
    return enc, pool


def _select_kernel(ku_ref, cu_ref, ckm_ref, cs_ref, use_ref, sel_ref,
                   cka_ref):
    N, K, _ = ku_ref.shape
    ku = ku_ref[...]
    cu = cu_ref[...]
    ck = jnp.sum(ku * cu[:, None, :], axis=2)
    valid = ckm_ref[...] != 0
    ckm = jnp.where(valid, ck, -1e9)
    mx = jnp.max(ckm, axis=1, keepdims=True)
    iota = lax.broadcasted_iota(jnp.int32, (N, K), 1)
    am = jnp.min(jnp.where(ckm == mx, iota, K), axis=1, keepdims=True)
    ids = jnp.where(use_ref[0, 0] != 0, cs_ref[...], am)
    iota_n = lax.broadcasted_iota(jnp.int32, (N, N), 1)
    oh = (ids == iota_n).astype(jnp.float32)
    sel_f = lax.dot_general(oh, ids.astype(jnp.float32),
                            (((1,), (0,)), ((), ())),
                            preferred_element_type=jnp.float32)
    sel_ref[...] = sel_f.astype(jnp.int32)
    cka_ref[...] = jnp.where(valid, ck, 0.0)


def _select(know_use, ctx_use, ck_mask_i, cs_ids2, use_flag):
    N, K, _ = know_use.shape
    sel, cka = pl.pallas_call(
        _select_kernel,
        grid=(1,),
        in_specs=[
            pl.BlockSpec((N, K, D), lambda i: (0, 0, 0)),
            pl.BlockSpec((N, D), lambda i: (0, 0)),
            pl.BlockSpec((N, K), lambda i: (0, 0)),
            pl.BlockSpec((N, 1), lambda i: (0, 0)),
            pl.BlockSpec((1, 1), lambda i: (0, 0)),
        ],
        out_specs=[
            pl.BlockSpec((N, 1), lambda i: (0, 0)),
            pl.BlockSpec((N, K), lambda i: (0, 0)),
        ],
        out_shape=[
            jax.ShapeDtypeStruct((N, 1), jnp.int32),
            jax.ShapeDtypeStruct((N, K), jnp.float32),
        ],
    )(know_use, ctx_use, ck_mask_i, cs_ids2, use_flag)
    return sel, cka


def _gc_kernel(sel_ref, kenc_ref, ktok_ref, cenc_ref, stok_ref, enc_ref,
               mask_ref):
    Tk = kenc_ref.shape[2]
    enc_ref[0, :Tk] = kenc_ref[0, 0]
    enc_ref[0, Tk:] = cenc_ref[0]
    mask_ref[0, 0, :Tk] = (ktok_ref[0, 0, 0] != 0).astype(jnp.int32)
    mask_ref[0, 0, Tk:] = (stok_ref[0, 0] != 0).astype(jnp.int32)


def _gather_concat(sel, know_enc4, know_tok4, ctx_enc, src_tok3):
    N, K, Tk, _ = know_enc4.shape
    Ts = ctx_enc.shape[1]
    Tf = Tk + Ts
    grid_spec = pltpu.PrefetchScalarGridSpec(
        num_scalar_prefetch=1,
        grid=(N,),
        in_specs=[
            pl.BlockSpec((1, 1, Tk, D), lambda n, s: (n, s[n], 0, 0)),
            pl.BlockSpec((1, 1, 1, Tk), lambda n, s: (n, s[n], 0, 0)),
            pl.BlockSpec((1, Ts, D), lambda n, s: (n, 0, 0)),
            pl.BlockSpec((1, 1, Ts), lambda n, s: (n, 0, 0)),
        ],
        out_specs=[
            pl.BlockSpec((1, Tf, D), lambda n, s: (n, 0, 0)),
            pl.BlockSpec((1, 1, Tf), lambda n, s: (n, 0, 0)),
        ],
    )
    enc, mask_i = pl.pallas_call(
        _gc_kernel,
        grid_spec=grid_spec,
        out_shape=[
            jax.ShapeDtypeStruct((N, Tf, D), jnp.float32),
            jax.ShapeDtypeStruct((N, 1, Tf), jnp.int32),
        ],
    )(sel, know_enc4, know_tok4, ctx_enc, src_tok3)
    return enc, mask_i


def _embed_rows(table, idx):
    B = idx.shape[0]
    info = plsc.get_sparse_core_info()
    NC, NS = info.num_cores, info.num_subcores
    NW = NC * NS
    b_per_w = B // NW
    CH = 64
    nch = b_per_w // CH

    mesh = plsc.VectorSubcoreMesh(core_axis_name="c", subcore_axis_name="s")

    @functools.partial(
        pl.kernel, mesh=mesh,
        out_type=jax.ShapeDtypeStruct((B, D), jnp.float32),
        scratch_types=[
            pltpu.VMEM((b_per_w,), jnp.int32),
            pltpu.VMEM((CH, D), jnp.float32),
            pltpu.VMEM((CH, D), jnp.float32),
            pltpu.SemaphoreType.DMA,
            pltpu.SemaphoreType.DMA,
            pltpu.SemaphoreType.DMA,
        ],
    )
    def gk(table_hbm, idx_hbm, out_hbm, idx_v, rows_a, rows_b, sem_a, sem_b,
           sem_out):
        wid = lax.axis_index("s") * NC + lax.axis_index("c")
        base = wid * b_per_w
        pltpu.sync_copy(idx_hbm.at[pl.ds(base, b_per_w)], idx_v)
        bufs = (rows_a, rows_b)
        sems = (sem_a, sem_b)
        copies = []
        for c in range(nch):
            copies.append(pltpu.async_copy(
                table_hbm.at[idx_v.at[pl.ds(c * CH, CH)]],
                bufs[c % 2], sems[c % 2]))
            if c > 0:
                copies[c - 1].wait()
                pltpu.async_copy(bufs[(c - 1) % 2],
                                 out_hbm.at[pl.ds(base + (c - 1) * CH, CH)],
                                 sem_out).wait()
        copies[nch - 1].wait()
        pltpu.async_copy(bufs[(nch - 1) % 2],
                         out_hbm.at[pl.ds(base + (nch - 1) * CH, CH)],
                         sem_out).wait()

    return gk(table, idx)


def kernel(src_tokens, know_tokens, ck_mask, cs_ids, use_cs_ids, params):
    p = params
    src = src_tokens.astype(jnp.int32)
    know = know_tokens.astype(jnp.int32)
    N, Ts = src.shape
    _, K, Tk = know.shape
    idx = jnp.concatenate([src.reshape(-1), know.reshape(-1)])
    rows = _embed_rows(p['emb'], idx)
    S_K = 4
    ctx_rows = rows[:N * Ts].reshape(N, Ts, D)
    know_rows = rows[N * Ts:].reshape(N * K // S_K, S_K * Tk, D)
    ctx_enc, ctx_pool = _encode(src.reshape(N, 1, Ts), ctx_rows,
                                p['pos'][:Ts], p, N, Ts, 1)
    know_enc, know_pool = _encode(
        know.reshape(N * K // S_K, 1, S_K * Tk), know_rows,
        p['pos'][:Tk], p, N * K, Tk, S_K)
    use_flag = jnp.asarray(use_cs_ids, jnp.int32).reshape(1, 1)
    sel2, cka = _select(know_pool.reshape(N, K, D), ctx_pool.reshape(N, D),
                        ck_mask.astype(jnp.int32), cs_ids.astype(
                            jnp.int32).reshape(N, 1), use_flag)
    full_enc, full_mask_i = _gather_concat(
        sel2.reshape(N), know_enc.reshape(N, K, Tk, D),
        know.reshape(N, K, 1, Tk), ctx_enc, src.reshape(N, 1, Ts))
    return full_enc, full_mask_i.reshape(N, Tk + Ts) != 0, cka


# speedup vs baseline: 2.5293x; 1.0984x over previous
import functools
import math

import jax
import jax.numpy as jnp
from jax import lax
from jax.experimental import pallas as pl
from jax.experimental.pallas import tpu as pltpu
from jax.experimental.pallas import tpu_sc as plsc

D = 512
H = 8
DH = 64
FF = 2048
SQRT_D = math.sqrt(512.0)


def _fdot(a, b):
    return jnp.dot(a, b, preferred_element_type=jnp.float32)


def _fdot_t(a, b):
    return lax.dot_general(a, b, (((1,), (1,)), ((), ())),
                           preferred_element_type=jnp.float32)


def _ln(x, g, b):
    m = jnp.mean(x, axis=1, keepdims=True)
    c = x - m
    v = jnp.mean(c * c, axis=1, keepdims=True)
    return c * lax.rsqrt(v + 1e-5) * g + b


def _enc_kernel(S, T, *refs):
    if S == 1:
        (tok_ref, emb_ref, pos_ref, wq_ref, wk_ref, wv_ref,
         wo_ref, w1_ref, b1_ref, w2_ref, b2_ref, g1_ref, be1_ref,
         g2_ref, be2_ref, enc_ref, pool_ref) = refs
        segb_ref = None
    else:
        (tok_ref, emb_ref, pos_ref, segb_ref, wq_ref, wk_ref, wv_ref,
         wo_ref, w1_ref, b1_ref, w2_ref, b2_ref, g1_ref, be1_ref,
         g2_ref, be2_ref, enc_ref, pool_ref) = refs
    R = S * T
    tok = tok_ref[0, 0]
    mask_row = (tok != 0).reshape(1, R)
    kb = jnp.where(mask_row, 0.0, -1e9)
    if S == 1:
        x = emb_ref[0] * SQRT_D + pos_ref[...]
        bias = kb
    else:
        x = (emb_ref[0].reshape(S, T, D) * SQRT_D
             + pos_ref[...][None]).reshape(R, D)
        bias = segb_ref[...] + kb
    h = _ln(x, g1_ref[...], be1_ref[...])
    q = _fdot(h, wq_ref[...]) * 0.125
    k = _fdot(h, wk_ref[...])
    v = _fdot(h, wv_ref[...])
    acc = x
    for hd in range(H):
        s = hd * DH
        sc = _fdot_t(q[:, s:s + DH], k[:, s:s + DH]) + bias
        ex = jnp.exp(sc)
        rs = 1.0 / jnp.sum(ex, axis=1, keepdims=True)
        oh = _fdot(ex, v[:, s:s + DH]) * rs
        acc = acc + _fdot(oh, wo_ref[s:s + DH, :])
    h2 = _ln(acc, g2_ref[...], be2_ref[...])
    f = jnp.maximum(_fdot(h2, w1_ref[...]) + b1_ref[...], 0.0)
    out = acc + _fdot(f, w2_ref[...]) + b2_ref[...]
    enc_m = out * jnp.transpose(mask_row.astype(jnp.float32))
    enc_ref[0] = enc_m
    lens = jnp.sum(mask_row.astype(jnp.float32).reshape(S, T), axis=1,
                   keepdims=True)
    pool = jnp.sum(enc_m.reshape(S, T, D), axis=1)
    pool_ref[0] = pool * lax.rsqrt(
        jnp.float32(D) * jnp.maximum(lens, 1.0))


def _encode(tokens3, emb_rows, pos_t, p, B, T, S):
    G = B // S
    R = S * T
    const = lambda i: (0, 0)
    body = functools.partial(_enc_kernel, S, T)
    seg_specs = []
    seg_ins = []
    if S > 1:
        seg_specs = [pl.BlockSpec((R, R), const)]
        ri = lax.broadcasted_iota(jnp.int32, (R, R), 0) // T
        ci = lax.broadcasted_iota(jnp.int32, (R, R), 1) // T
        seg_ins = [jnp.where(ri == ci, 0.0, -1e9).astype(jnp.float32)]
    enc, pool = pl.pallas_call(
        body,
        grid=(G,),
        in_specs=[
            pl.BlockSpec((1, 1, R), lambda i: (i, 0, 0)),
            pl.BlockSpec((1, R, D), lambda i: (i, 0, 0)),
            pl.BlockSpec((T, D), const),
        ] + seg_specs + [
            pl.BlockSpec((D, D), const),
            pl.BlockSpec((D, D), const),
            pl.BlockSpec((D, D), const),
            pl.BlockSpec((D, D), const),
            pl.BlockSpec((D, FF), const),
            pl.BlockSpec((1, FF), const),
            pl.BlockSpec((FF, D), const),
            pl.BlockSpec((1, D), const),
            pl.BlockSpec((1, D), const),
            pl.BlockSpec((1, D), const),
            pl.BlockSpec((1, D), const),
            pl.BlockSpec((1, D), const),
        ],
        out_specs=[
            pl.BlockSpec((1, R, D), lambda i: (i, 0, 0)),
            pl.BlockSpec((1, S, D), lambda i: (i, 0, 0)),
        ],
        out_shape=[
            jax.ShapeDtypeStruct((G, R, D), jnp.float32),
            jax.ShapeDtypeStruct((G, S, D), jnp.float32),
        ],
    )(tokens3, emb_rows, pos_t, *seg_ins,
      p['Wq'], p['Wk'], p['Wv'], p['Wo'],
      p['W1'], p['b1'].reshape(1, FF), p['W2'], p['b2'].reshape(1, D),
      p['g1'].reshape(1, D), p['be1'].reshape(1, D),
      p['g2'].reshape(1, D), p['be2'].reshape(1, D))
    return enc, pool


def _select_kernel(ku_ref, cu_ref, ckm_ref, cs_ref, use_ref, sel_ref,
                   cka_ref):
    N, K, _ = ku_ref.shape
    ku = ku_ref[...]
    cu = cu_ref[...]
    ck = jnp.sum(ku * cu[:, None, :], axis=2)
    valid = ckm_ref[...] != 0
    ckm = jnp.where(valid, ck, -1e9)
    mx = jnp.max(ckm, axis=1, keepdims=True)
    iota = lax.broadcasted_iota(jnp.int32, (N, K), 1)
    am = jnp.min(jnp.where(ckm == mx, iota, K), axis=1, keepdims=True)
    ids = jnp.where(use_ref[0, 0] != 0, cs_ref[...], am)
    iota_n = lax.broadcasted_iota(jnp.int32, (N, N), 1)
    oh = (ids == iota_n).astype(jnp.float32)
    sel_f = lax.dot_general(oh, ids.astype(jnp.float32),
                            (((1,), (0,)), ((), ())),
                            preferred_element_type=jnp.float32)
    sel_ref[...] = sel_f.astype(jnp.int32)
    cka_ref[...] = jnp.where(valid, ck, 0.0)


def _select(know_use, ctx_use, ck_mask_i, cs_ids2, use_flag):
    N, K, _ = know_use.shape
    sel, cka = pl.pallas_call(
        _select_kernel,
        grid=(1,),
        in_specs=[
            pl.BlockSpec((N, K, D), lambda i: (0, 0, 0)),
            pl.BlockSpec((N, D), lambda i: (0, 0)),
            pl.BlockSpec((N, K), lambda i: (0, 0)),
            pl.BlockSpec((N, 1), lambda i: (0, 0)),
            pl.BlockSpec((1, 1), lambda i: (0, 0)),
        ],
        out_specs=[
            pl.BlockSpec((N, 1), lambda i: (0, 0)),
            pl.BlockSpec((N, K), lambda i: (0, 0)),
        ],
        out_shape=[
            jax.ShapeDtypeStruct((N, 1), jnp.int32),
            jax.ShapeDtypeStruct((N, K), jnp.float32),
        ],
    )(know_use, ctx_use, ck_mask_i, cs_ids2, use_flag)
    return sel, cka


def _gc_kernel(sel_ref, kenc_ref, ktok_ref, cenc_ref, stok_ref, enc_ref,
               mask_ref):
    Tk = kenc_ref.shape[2]
    enc_ref[0, :Tk] = kenc_ref[0, 0]
    enc_ref[0, Tk:] = cenc_ref[0]
    mask_ref[0, 0, :Tk] = (ktok_ref[0, 0, 0] != 0).astype(jnp.int32)
    mask_ref[0, 0, Tk:] = (stok_ref[0, 0] != 0).astype(jnp.int32)


def _gather_concat(sel, know_enc4, know_tok4, ctx_enc, src_tok3):
    N, K, Tk, _ = know_enc4.shape
    Ts = ctx_enc.shape[1]
    Tf = Tk + Ts
    grid_spec = pltpu.PrefetchScalarGridSpec(
        num_scalar_prefetch=1,
        grid=(N,),
        in_specs=[
            pl.BlockSpec((1, 1, Tk, D), lambda n, s: (n, s[n], 0, 0)),
            pl.BlockSpec((1, 1, 1, Tk), lambda n, s: (n, s[n], 0, 0)),
            pl.BlockSpec((1, Ts, D), lambda n, s: (n, 0, 0)),
            pl.BlockSpec((1, 1, Ts), lambda n, s: (n, 0, 0)),
        ],
        out_specs=[
            pl.BlockSpec((1, Tf, D), lambda n, s: (n, 0, 0)),
            pl.BlockSpec((1, 1, Tf), lambda n, s: (n, 0, 0)),
        ],
    )
    enc, mask_i = pl.pallas_call(
        _gc_kernel,
        grid_spec=grid_spec,
        out_shape=[
            jax.ShapeDtypeStruct((N, Tf, D), jnp.float32),
            jax.ShapeDtypeStruct((N, 1, Tf), jnp.int32),
        ],
    )(sel, know_enc4, know_tok4, ctx_enc, src_tok3)
    return enc, mask_i


def _embed_rows(table, idx):
    B = idx.shape[0]
    info = plsc.get_sparse_core_info()
    NC, NS = info.num_cores, info.num_subcores
    NW = NC * NS
    b_per_w = B // NW
    CH = 64
    nch = b_per_w // CH

    NBUF = 3
    mesh = plsc.VectorSubcoreMesh(core_axis_name="c", subcore_axis_name="s")

    @functools.partial(
        pl.kernel, mesh=mesh,
        out_type=jax.ShapeDtypeStruct((B, D), jnp.float32),
        scratch_types=(
            [pltpu.VMEM((b_per_w,), jnp.int32)]
            + [pltpu.VMEM((CH, D), jnp.float32)] * NBUF
            + [pltpu.SemaphoreType.DMA] * (2 * NBUF)
        ),
    )
    def gk(table_hbm, idx_hbm, out_hbm, idx_v, *bufs_sems):
        bufs = bufs_sems[:NBUF]
        gsems = bufs_sems[NBUF:2 * NBUF]
        osems = bufs_sems[2 * NBUF:]
        wid = lax.axis_index("s") * NC + lax.axis_index("c")
        base = wid * b_per_w
        pltpu.sync_copy(idx_hbm.at[pl.ds(base, b_per_w)], idx_v)
        gcopies = [None] * nch
        ocopies = [None] * nch
        for c in range(nch):
            if c >= NBUF:
                ocopies[c - NBUF].wait()
            gcopies[c] = pltpu.async_copy(
                table_hbm.at[idx_v.at[pl.ds(c * CH, CH)]],
                bufs[c % NBUF], gsems[c % NBUF])
            if c >= 1:
                gcopies[c - 1].wait()
                ocopies[c - 1] = pltpu.async_copy(
                    bufs[(c - 1) % NBUF],
                    out_hbm.at[pl.ds(base + (c - 1) * CH, CH)],
                    osems[(c - 1) % NBUF])
        gcopies[nch - 1].wait()
        ocopies[nch - 1] = pltpu.async_copy(
            bufs[(nch - 1) % NBUF],
            out_hbm.at[pl.ds(base + (nch - 1) * CH, CH)],
            osems[(nch - 1) % NBUF])
        for c in range(max(0, nch - NBUF), nch):
            if ocopies[c] is not None and c >= nch - NBUF:
                ocopies[c].wait()

    return gk(table, idx)


def kernel(src_tokens, know_tokens, ck_mask, cs_ids, use_cs_ids, params):
    p = params
    src = src_tokens.astype(jnp.int32)
    know = know_tokens.astype(jnp.int32)
    N, Ts = src.shape
    _, K, Tk = know.shape
    S_K = 4
    know_rows = _embed_rows(p['emb'], know.reshape(-1)).reshape(
        N * K // S_K, S_K * Tk, D)
    ctx_rows = _embed_rows(p['emb'], src.reshape(-1)).reshape(N, Ts, D)
    ctx_enc, ctx_pool = _encode(src.reshape(N, 1, Ts), ctx_rows,
                                p['pos'][:Ts], p, N, Ts, 1)
    know_enc, know_pool = _encode(
        know.reshape(N * K // S_K, 1, S_K * Tk), know_rows,
        p['pos'][:Tk], p, N * K, Tk, S_K)
    use_flag = jnp.asarray(use_cs_ids, jnp.int32).reshape(1, 1)
    sel2, cka = _select(know_pool.reshape(N, K, D), ctx_pool.reshape(N, D),
                        ck_mask.astype(jnp.int32), cs_ids.astype(
                            jnp.int32).reshape(N, 1), use_flag)
    full_enc, full_mask_i = _gather_concat(
        sel2.reshape(N), know_enc.reshape(N, K, Tk, D),
        know.reshape(N, K, 1, Tk), ctx_enc, src.reshape(N, 1, Ts))
    return full_enc, full_mask_i.reshape(N, Tk + Ts) != 0, cka


# speedup vs baseline: 3.0567x; 1.2085x over previous
import functools
import math

import jax
import jax.numpy as jnp
from jax import lax
from jax.experimental import pallas as pl
from jax.experimental.pallas import tpu as pltpu
from jax.experimental.pallas import tpu_sc as plsc

D = 512
H = 8
DH = 64
FF = 2048
SQRT_D = math.sqrt(512.0)


def _fdot(a, b):
    return jnp.dot(a, b, preferred_element_type=jnp.float32)


def _fdot_t(a, b):
    return lax.dot_general(a, b, (((1,), (1,)), ((), ())),
                           preferred_element_type=jnp.float32)


def _ln(x, g, b):
    m = jnp.mean(x, axis=1, keepdims=True)
    c = x - m
    v = jnp.mean(c * c, axis=1, keepdims=True)
    return c * lax.rsqrt(v + 1e-5) * g + b


def _enc_kernel(S, T, *refs):
    if S == 1:
        (tok_ref, emb_ref, pos_ref, wq_ref, wk_ref, wv_ref,
         wo_ref, w1_ref, b1_ref, w2_ref, b2_ref, g1_ref, be1_ref,
         g2_ref, be2_ref, enc_ref, pool_ref) = refs
        segb_ref = None
    else:
        (tok_ref, emb_ref, pos_ref, segb_ref, wq_ref, wk_ref, wv_ref,
         wo_ref, w1_ref, b1_ref, w2_ref, b2_ref, g1_ref, be1_ref,
         g2_ref, be2_ref, enc_ref, pool_ref) = refs
    R = S * T
    tok = tok_ref[0, 0]
    mask_row = (tok != 0).reshape(1, R)
    kb = jnp.where(mask_row, 0.0, -1e9)
    if S == 1:
        x = emb_ref[0] * SQRT_D + pos_ref[...]
        bias = kb
    else:
        x = (emb_ref[0].reshape(S, T, D) * SQRT_D
             + pos_ref[...][None]).reshape(R, D)
        bias = segb_ref[...] + kb
    h = _ln(x, g1_ref[...], be1_ref[...])
    q = _fdot(h, wq_ref[...]) * 0.125
    k = _fdot(h, wk_ref[...])
    v = _fdot(h, wv_ref[...])
    R2 = q.shape[0]
    ones_col = jnp.ones((R2, 1), jnp.float32)
    ohs = []
    for hd in range(H):
        s = hd * DH
        sc = _fdot_t(q[:, s:s + DH], k[:, s:s + DH]) + bias
        ex = jnp.exp(sc)
        va = jnp.concatenate([v[:, s:s + DH], ones_col], axis=1)
        oa = _fdot(ex, va)
        ohs.append(oa[:, :DH] * (1.0 / oa[:, DH:DH + 1]))
    acc = x + _fdot(jnp.concatenate(ohs, axis=1), wo_ref[...])
    h2 = _ln(acc, g2_ref[...], be2_ref[...])
    f = jnp.maximum(_fdot(h2, w1_ref[...]) + b1_ref[...], 0.0)
    out = acc + _fdot(f, w2_ref[...]) + b2_ref[...]
    enc_m = out * jnp.transpose(mask_row.astype(jnp.float32))
    enc_ref[0] = enc_m
    lens = jnp.sum(mask_row.astype(jnp.float32).reshape(S, T), axis=1,
                   keepdims=True)
    pool = jnp.sum(enc_m.reshape(S, T, D), axis=1)
    pool_ref[0] = pool * lax.rsqrt(
        jnp.float32(D) * jnp.maximum(lens, 1.0))


def _encode(tokens3, emb_rows, pos_t, p, B, T, S):
    G = B // S
    R = S * T
    const = lambda i: (0, 0)
    body = functools.partial(_enc_kernel, S, T)
    seg_specs = []
    seg_ins = []
    if S > 1:
        seg_specs = [pl.BlockSpec((R, R), const)]
        ri = lax.broadcasted_iota(jnp.int32, (R, R), 0) // T
        ci = lax.broadcasted_iota(jnp.int32, (R, R), 1) // T
        seg_ins = [jnp.where(ri == ci, 0.0, -1e9).astype(jnp.float32)]
    enc, pool = pl.pallas_call(
        body,
        grid=(G,),
        in_specs=[
            pl.BlockSpec((1, 1, R), lambda i: (i, 0, 0)),
            pl.BlockSpec((1, R, D), lambda i: (i, 0, 0)),
            pl.BlockSpec((T, D), const),
        ] + seg_specs + [
            pl.BlockSpec((D, D), const),
            pl.BlockSpec((D, D), const),
            pl.BlockSpec((D, D), const),
            pl.BlockSpec((D, D), const),
            pl.BlockSpec((D, FF), const),
            pl.BlockSpec((1, FF), const),
            pl.BlockSpec((FF, D), const),
            pl.BlockSpec((1, D), const),
            pl.BlockSpec((1, D), const),
            pl.BlockSpec((1, D), const),
            pl.BlockSpec((1, D), const),
            pl.BlockSpec((1, D), const),
        ],
        out_specs=[
            pl.BlockSpec((1, R, D), lambda i: (i, 0, 0)),
            pl.BlockSpec((1, S, D), lambda i: (i, 0, 0)),
        ],
        out_shape=[
            jax.ShapeDtypeStruct((G, R, D), jnp.float32),
            jax.ShapeDtypeStruct((G, S, D), jnp.float32),
        ],
    )(tokens3, emb_rows, pos_t, *seg_ins,
      p['Wq'], p['Wk'], p['Wv'], p['Wo'],
      p['W1'], p['b1'].reshape(1, FF), p['W2'], p['b2'].reshape(1, D),
      p['g1'].reshape(1, D), p['be1'].reshape(1, D),
      p['g2'].reshape(1, D), p['be2'].reshape(1, D))
    return enc, pool


def _select_kernel(ku_ref, cu_ref, ckm_ref, cs_ref, use_ref, sel_ref,
                   cka_ref):
    N, K, _ = ku_ref.shape
    ku = ku_ref[...]
    cu = cu_ref[...]
    ck = jnp.sum(ku * cu[:, None, :], axis=2)
    valid = ckm_ref[...] != 0
    ckm = jnp.where(valid, ck, -1e9)
    mx = jnp.max(ckm, axis=1, keepdims=True)
    iota = lax.broadcasted_iota(jnp.int32, (N, K), 1)
    am = jnp.min(jnp.where(ckm == mx, iota, K), axis=1, keepdims=True)
    ids = jnp.where(use_ref[0, 0] != 0, cs_ref[...], am)
    iota_n = lax.broadcasted_iota(jnp.int32, (N, N), 1)
    oh = (ids == iota_n).astype(jnp.float32)
    sel_f = lax.dot_general(oh, ids.astype(jnp.float32),
                            (((1,), (0,)), ((), ())),
                            preferred_element_type=jnp.float32)
    sel_ref[...] = sel_f.astype(jnp.int32)
    cka_ref[...] = jnp.where(valid, ck, 0.0)


def _select(know_use, ctx_use, ck_mask_i, cs_ids2, use_flag):
    N, K, _ = know_use.shape
    sel, cka = pl.pallas_call(
        _select_kernel,
        grid=(1,),
        in_specs=[
            pl.BlockSpec((N, K, D), lambda i: (0, 0, 0)),
            pl.BlockSpec((N, D), lambda i: (0, 0)),
            pl.BlockSpec((N, K), lambda i: (0, 0)),
            pl.BlockSpec((N, 1), lambda i: (0, 0)),
            pl.BlockSpec((1, 1), lambda i: (0, 0)),
        ],
        out_specs=[
            pl.BlockSpec((N, 1), lambda i: (0, 0)),
            pl.BlockSpec((N, K), lambda i: (0, 0)),
        ],
        out_shape=[
            jax.ShapeDtypeStruct((N, 1), jnp.int32),
            jax.ShapeDtypeStruct((N, K), jnp.float32),
        ],
    )(know_use, ctx_use, ck_mask_i, cs_ids2, use_flag)
    return sel, cka


def _gc_kernel(sel_ref, kenc_ref, ktok_ref, cenc_ref, stok_ref, enc_ref,
               mask_ref):
    Tk = kenc_ref.shape[2]
    enc_ref[0, :Tk] = kenc_ref[0, 0]
    enc_ref[0, Tk:] = cenc_ref[0]
    mask_ref[0, 0, :Tk] = (ktok_ref[0, 0, 0] != 0).astype(jnp.int32)
    mask_ref[0, 0, Tk:] = (stok_ref[0, 0] != 0).astype(jnp.int32)


def _gather_concat(sel, know_enc4, know_tok4, ctx_enc, src_tok3):
    N, K, Tk, _ = know_enc4.shape
    Ts = ctx_enc.shape[1]
    Tf = Tk + Ts
    grid_spec = pltpu.PrefetchScalarGridSpec(
        num_scalar_prefetch=1,
        grid=(N,),
        in_specs=[
            pl.BlockSpec((1, 1, Tk, D), lambda n, s: (n, s[n], 0, 0)),
            pl.BlockSpec((1, 1, 1, Tk), lambda n, s: (n, s[n], 0, 0)),
            pl.BlockSpec((1, Ts, D), lambda n, s: (n, 0, 0)),
            pl.BlockSpec((1, 1, Ts), lambda n, s: (n, 0, 0)),
        ],
        out_specs=[
            pl.BlockSpec((1, Tf, D), lambda n, s: (n, 0, 0)),
            pl.BlockSpec((1, 1, Tf), lambda n, s: (n, 0, 0)),
        ],
    )
    enc, mask_i = pl.pallas_call(
        _gc_kernel,
        grid_spec=grid_spec,
        out_shape=[
            jax.ShapeDtypeStruct((N, Tf, D), jnp.float32),
            jax.ShapeDtypeStruct((N, 1, Tf), jnp.int32),
        ],
    )(sel, know_enc4, know_tok4, ctx_enc, src_tok3)
    return enc, mask_i


def _embed_rows(table, idx):
    B = idx.shape[0]
    info = plsc.get_sparse_core_info()
    NC, NS = info.num_cores, info.num_subcores
    NW = NC * NS
    b_per_w = B // NW
    CH = 64
    nch = b_per_w // CH

    NBUF = 3
    mesh = plsc.VectorSubcoreMesh(core_axis_name="c", subcore_axis_name="s")

    @functools.partial(
        pl.kernel, mesh=mesh,
        out_type=jax.ShapeDtypeStruct((B, D), jnp.float32),
        scratch_types=(
            [pltpu.VMEM((b_per_w,), jnp.int32)]
            + [pltpu.VMEM((CH, D), jnp.float32)] * NBUF
            + [pltpu.SemaphoreType.DMA] * (2 * NBUF)
        ),
    )
    def gk(table_hbm, idx_hbm, out_hbm, idx_v, *bufs_sems):
        bufs = bufs_sems[:NBUF]
        gsems = bufs_sems[NBUF:2 * NBUF]
        osems = bufs_sems[2 * NBUF:]
        wid = lax.axis_index("s") * NC + lax.axis_index("c")
        base = wid * b_per_w
        pltpu.sync_copy(idx_hbm.at[pl.ds(base, b_per_w)], idx_v)
        gcopies = [None] * nch
        ocopies = [None] * nch
        for c in range(nch):
            if c >= NBUF:
                ocopies[c - NBUF].wait()
            gcopies[c] = pltpu.async_copy(
                table_hbm.at[idx_v.at[pl.ds(c * CH, CH)]],
                bufs[c % NBUF], gsems[c % NBUF])
            if c >= 1:
                gcopies[c - 1].wait()
                ocopies[c - 1] = pltpu.async_copy(
                    bufs[(c - 1) % NBUF],
                    out_hbm.at[pl.ds(base + (c - 1) * CH, CH)],
                    osems[(c - 1) % NBUF])
        gcopies[nch - 1].wait()
        ocopies[nch - 1] = pltpu.async_copy(
            bufs[(nch - 1) % NBUF],
            out_hbm.at[pl.ds(base + (nch - 1) * CH, CH)],
            osems[(nch - 1) % NBUF])
        for c in range(max(0, nch - NBUF), nch):
            if ocopies[c] is not None and c >= nch - NBUF:
                ocopies[c].wait()

    return gk(table, idx)


def kernel(src_tokens, know_tokens, ck_mask, cs_ids, use_cs_ids, params):
    p = params
    src = src_tokens.astype(jnp.int32)
    know = know_tokens.astype(jnp.int32)
    N, Ts = src.shape
    _, K, Tk = know.shape
    S_K = 4
    ctx_rows = _embed_rows(p['emb'], src.reshape(-1)).reshape(N, Ts, D)
    know_rows = _embed_rows(p['emb'], know.reshape(-1)).reshape(
        N * K // S_K, S_K * Tk, D)
    ctx_enc, ctx_pool = _encode(src.reshape(N, 1, Ts), ctx_rows,
                                p['pos'][:Ts], p, N, Ts, 1)
    know_enc, know_pool = _encode(
        know.reshape(N * K // S_K, 1, S_K * Tk), know_rows,
        p['pos'][:Tk], p, N * K, Tk, S_K)
    use_flag = jnp.asarray(use_cs_ids, jnp.int32).reshape(1, 1)
    sel2, cka = _select(know_pool.reshape(N, K, D), ctx_pool.reshape(N, D),
                        ck_mask.astype(jnp.int32), cs_ids.astype(
                            jnp.int32).reshape(N, 1), use_flag)
    full_enc, full_mask_i = _gather_concat(
        sel2.reshape(N), know_enc.reshape(N, K, Tk, D),
        know.reshape(N, K, 1, Tk), ctx_enc, src.reshape(N, 1, Ts))
    return full_enc, full_mask_i.reshape(N, Tk + Ts) != 0, cka
